# baseline (speedup 1.0000x reference)
import jax
import jax.numpy as jnp
import numpy as np
from jax.experimental import pallas as pl
from jax.experimental.pallas import tpu as pltpu

_ROWS = 128
_CHUNK = 4096
_SUB = 128
_NS = _CHUNK // _SUB

_TRI = np.triu(np.ones((_SUB, _SUB), np.float32)).astype(jnp.bfloat16)
_OBLK = (np.arange(_CHUNK)[:, None] // _SUB ==
         np.arange(_NS)[None, :]).astype(jnp.bfloat16)
_TEXF = np.concatenate(
    [(np.arange(_NS)[:, None] < (np.arange(_CHUNK)[None, :] // _SUB)),
     np.ones((1, _CHUNK), np.bool_)], axis=0).astype(jnp.bfloat16)


def _dot(a, b):
    return jax.lax.dot_general(
        a, b, (((1,), (0,)), ((), ())), preferred_element_type=jnp.float32)


def _body(x_ref, m_ref, tri_ref, oblk_ref, texf_ref, o_ref, carry_ref):
    i = pl.program_id(0)

    @pl.when(i == 0)
    def _init():
        carry_ref[...] = jnp.zeros_like(carry_ref)

    xm = jnp.where(m_ref[...], x_ref[...], 0.0).astype(jnp.bfloat16)
    tri = tri_ref[...]
    oblk = oblk_ref[...]
    texf = texf_ref[...]

    t = _dot(xm, oblk)
    tc = jnp.concatenate(
        [t, carry_ref[:, 0:1]], axis=1).astype(jnp.bfloat16)
    p = _dot(tc, texf)
    for s in range(_NS):
        y = _dot(xm[:, s * _SUB:(s + 1) * _SUB], tri)
        o_ref[:, s * _SUB:(s + 1) * _SUB] = y + p[:, s * _SUB:(s + 1) * _SUB]
    carry_ref[:, 0:1] = p[:, _CHUNK - 1:_CHUNK] + t[:, _NS - 1:_NS]


def kernel(x, mask):
    n = x.shape[1]
    grid = (n // _CHUNK,)
    spec = pl.BlockSpec((_ROWS, _CHUNK), lambda i: (0, i))

    def _const_spec(shape):
        return pl.BlockSpec(shape, lambda i: (0, 0))

    return pl.pallas_call(
        _body,
        grid=grid,
        in_specs=[spec, spec, _const_spec(_TRI.shape),
                  _const_spec(_OBLK.shape), _const_spec(_TEXF.shape)],
        out_specs=spec,
        out_shape=jax.ShapeDtypeStruct(x.shape, x.dtype),
        scratch_shapes=[pltpu.VMEM((_ROWS, 128), jnp.float32)],
    )(x, mask, jnp.asarray(_TRI), jnp.asarray(_OBLK), jnp.asarray(_TEXF))


# speedup vs baseline: 1.0426x; 1.0426x over previous
import jax
import jax.numpy as jnp
import numpy as np
from jax.experimental import pallas as pl
from jax.experimental.pallas import tpu as pltpu

_ROWS = 128
_CHUNK = 8192
_SUB = 128
_NS = _CHUNK // _SUB

_TRI = np.triu(np.ones((_SUB, _SUB), np.float32)).astype(jnp.bfloat16)
_OBLK = (np.arange(_CHUNK)[:, None] // _SUB ==
         np.arange(_NS)[None, :]).astype(jnp.bfloat16)
_TEXF = np.concatenate(
    [(np.arange(_NS)[:, None] < (np.arange(_CHUNK)[None, :] // _SUB)),
     np.ones((1, _CHUNK), np.bool_)], axis=0).astype(jnp.bfloat16)


def _dot(a, b):
    return jax.lax.dot_general(
        a, b, (((1,), (0,)), ((), ())), preferred_element_type=jnp.float32)


def _body(x_ref, m_ref, tri_ref, oblk_ref, texf_ref, o_ref, carry_ref):
    i = pl.program_id(0)

    @pl.when(i == 0)
    def _init():
        carry_ref[...] = jnp.zeros_like(carry_ref)

    xm = jnp.where(m_ref[...], x_ref[...], 0.0).astype(jnp.bfloat16)
    tri = tri_ref[...]
    oblk = oblk_ref[...]
    texf = texf_ref[...]

    t = _dot(xm, oblk)
    tc = jnp.concatenate(
        [t, carry_ref[:, 0:1]], axis=1).astype(jnp.bfloat16)
    last = None
    for s in range(_NS):
        y = _dot(xm[:, s * _SUB:(s + 1) * _SUB], tri)
        y = y + _dot(tc, texf[:, s * _SUB:(s + 1) * _SUB])
        o_ref[:, s * _SUB:(s + 1) * _SUB] = y
        last = y
    carry_ref[:, 0:1] = last[:, _SUB - 1:_SUB]


def kernel(x, mask):
    n = x.shape[1]
    grid = (n // _CHUNK,)
    spec = pl.BlockSpec((_ROWS, _CHUNK), lambda i: (0, i))

    def _const_spec(shape):
        return pl.BlockSpec(shape, lambda i: (0, 0))

    return pl.pallas_call(
        _body,
        grid=grid,
        in_specs=[spec, spec, _const_spec(_TRI.shape),
                  _const_spec(_OBLK.shape), _const_spec(_TEXF.shape)],
        out_specs=spec,
        out_shape=jax.ShapeDtypeStruct(x.shape, x.dtype),
        scratch_shapes=[pltpu.VMEM((_ROWS, 128), jnp.float32)],
    )(x, mask, jnp.asarray(_TRI), jnp.asarray(_OBLK), jnp.asarray(_TEXF))


# speedup vs baseline: 1.0490x; 1.0061x over previous
import jax
import jax.numpy as jnp
import numpy as np
from jax.experimental import pallas as pl
from jax.experimental.pallas import tpu as pltpu

_ROWS = 128
_CHUNK = 8192
_SUB = 128
_NS = _CHUNK // _SUB

_TRI = np.triu(np.ones((_SUB, _SUB), np.float32)).astype(jnp.bfloat16)
_OBLK = (np.arange(_CHUNK)[:, None] // _SUB ==
         np.arange(_NS)[None, :]).astype(jnp.bfloat16)
_TEXF = np.concatenate(
    [(np.arange(_NS)[:, None] < (np.arange(_CHUNK)[None, :] // _SUB)),
     np.ones((1, _CHUNK), np.bool_)], axis=0).astype(jnp.bfloat16)


def _dot(a, b):
    return jax.lax.dot_general(
        a, b, (((1,), (0,)), ((), ())), preferred_element_type=jnp.float32)


def _body(x_ref, m_ref, tri_ref, oblk_ref, texf_ref, o_ref, carry_ref):
    i = pl.program_id(0)

    @pl.when(i == 0)
    def _init():
        carry_ref[...] = jnp.zeros_like(carry_ref)

    xm = jnp.where(m_ref[...], x_ref[...], 0.0).astype(jnp.bfloat16)
    tri = tri_ref[...]
    oblk = oblk_ref[...]
    texf = texf_ref[...]

    t = _dot(xm, oblk)
    tc = jnp.concatenate(
        [t, carry_ref[:, 0:1]], axis=1).astype(jnp.bfloat16)
    p = _dot(tc, texf)
    for s in range(_NS):
        y = _dot(xm[:, s * _SUB:(s + 1) * _SUB], tri)
        o_ref[:, s * _SUB:(s + 1) * _SUB] = y + p[:, s * _SUB:(s + 1) * _SUB]
    carry_ref[:, 0:1] = p[:, _CHUNK - 1:_CHUNK] + t[:, _NS - 1:_NS]


def kernel(x, mask):
    n = x.shape[1]
    grid = (n // _CHUNK,)
    spec = pl.BlockSpec((_ROWS, _CHUNK), lambda i: (0, i))

    def _const_spec(shape):
        return pl.BlockSpec(shape, lambda i: (0, 0))

    return pl.pallas_call(
        _body,
        grid=grid,
        in_specs=[spec, spec, _const_spec(_TRI.shape),
                  _const_spec(_OBLK.shape), _const_spec(_TEXF.shape)],
        out_specs=spec,
        out_shape=jax.ShapeDtypeStruct(x.shape, x.dtype),
        scratch_shapes=[pltpu.VMEM((_ROWS, 128), jnp.float32)],
    )(x, mask, jnp.asarray(_TRI), jnp.asarray(_OBLK), jnp.asarray(_TEXF))
